# baseline (speedup 1.0000x reference)
import functools

import jax
import jax.numpy as jnp
import numpy as np
from jax import lax
from jax.experimental import pallas as pl
from jax.experimental.pallas import tpu as pltpu
from jax.experimental.pallas import tpu_sc as plsc

NUM_LEVELS = 16
BASE_RES = 16
MAX_RES = 2048
LOG2_T = 19
T = 1 << LOG2_T
N_POINTS = 262144
HIDDEN = 32
F_IN = NUM_LEVELS * 2

_B = np.exp((np.log(MAX_RES) - np.log(BASE_RES)) / (NUM_LEVELS - 1))
_RES = [int(np.floor(BASE_RES * (_B ** l))) for l in range(NUM_LEVELS)]

_P1 = np.uint32(2654435761)
_P2 = np.uint32(805459861)
_MASK = np.uint32(T - 1)

NC = 2
NS = 16
NW = NC * NS
PPW = N_POINTS // NW
C = 512
NCHUNK = PPW // C
G = C // 16
NIDX = 8 * C
NDMA = NIDX // 128


def _sc_body(x_hbm, tab_hbm, out_hbm, xbuf, idx0, idx1, rows0, rows1,
             featbuf, sem0, sem1):
    wid = lax.axis_index("s") * NC + lax.axis_index("c")
    lane = lax.iota(jnp.int32, 16)
    col0 = lane * 0
    col1 = col0 + 1
    col2 = col0 + 2
    idxbufs = (idx0, idx1)
    rowbufs = (rows0, rows1)
    sems = (sem0, sem1)

    def load_xyz(g):
        r16 = g * 16 + lane
        x0 = plsc.load_gather(xbuf, [r16, col0])
        x1 = plsc.load_gather(xbuf, [r16, col1])
        x2 = plsc.load_gather(xbuf, [r16, col2])
        return r16, x0, x1, x2

    def compute_idx(l, ib):
        res = np.float32(_RES[l])
        off = np.int32(l * T)

        def g_body(g, c_):
            _, x0, x1, x2 = load_xyz(g)
            i0 = (x0 * res).astype(jnp.int32)
            i1 = (x1 * res).astype(jnp.int32)
            i2 = (x2 * res).astype(jnp.int32)
            hx0 = i0.astype(jnp.uint32)
            hx1 = hx0 + np.uint32(1)
            u1 = i1.astype(jnp.uint32)
            u2 = i2.astype(jnp.uint32)
            ty0 = u1 * _P1
            ty1 = (u1 + np.uint32(1)) * _P1
            tz0 = u2 * _P2
            tz1 = (u2 + np.uint32(1)) * _P2
            grow = jnp.full((16,), g, jnp.int32)
            lane8 = lane * 8
            for c in range(8):
                hx = hx1 if (c & 1) else hx0
                ty = ty1 if (c & 2) else ty0
                tz = tz1 if (c & 4) else tz0
                h = ((hx ^ ty ^ tz) & _MASK).astype(jnp.int32) + off
                plsc.store_scatter(ib, [grow, lane8 + c], h)
            return c_

        lax.fori_loop(0, G, g_body, 0)

    def fire(ib, rb, sem):
        def j_body(j, c_):
            pltpu.async_copy(tab_hbm.at[ib.at[j]],
                             rb.at[pl.ds(j * 128, 128), :], sem)
            return c_

        lax.fori_loop(0, NDMA, j_body, 0)

    def drain(ib, rb, sem):
        def j_body(j, c_):
            pltpu.make_async_copy(tab_hbm.at[ib.at[j]],
                                  rb.at[pl.ds(j * 128, 128), :], sem).wait()
            return c_

        lax.fori_loop(0, NDMA, j_body, 0)

    def compute_feat(l, rb):
        res = np.float32(_RES[l])
        fcol0 = col0 + (2 * l)
        fcol1 = col0 + (2 * l + 1)

        def g_body(g, c_):
            r16, x0, x1, x2 = load_xyz(g)
            p0 = x0 * res
            p1 = x1 * res
            p2 = x2 * res
            i0 = p0.astype(jnp.int32)
            i1 = p1.astype(jnp.int32)
            i2 = p2.astype(jnp.int32)
            fx = p0 - i0.astype(jnp.float32)
            fy = p1 - i1.astype(jnp.float32)
            fz = p2 - i2.astype(jnp.float32)
            wx = (1.0 - fx, fx)
            wy = (1.0 - fy, fy)
            wz = (1.0 - fz, fz)
            wxy = [wx[cx] * wy[cy] for cy in range(2) for cx in range(2)]
            acc0 = None
            acc1 = None
            rbase = r16 * 8
            for c in range(8):
                w = wxy[c & 3] * wz[(c >> 2) & 1]
                ridx = rbase + c
                f0 = plsc.load_gather(rb, [ridx, col0])
                f1 = plsc.load_gather(rb, [ridx, col1])
                if c == 0:
                    acc0 = w * f0
                    acc1 = w * f1
                else:
                    acc0 = acc0 + w * f0
                    acc1 = acc1 + w * f1
            plsc.store_scatter(featbuf, [r16, fcol0], acc0)
            plsc.store_scatter(featbuf, [r16, fcol1], acc1)
            return c_

        lax.fori_loop(0, G, g_body, 0)

    def chunk_body(ci, c_):
        base = wid * PPW + ci * C
        pltpu.sync_copy(x_hbm.at[pl.ds(base, C), :], xbuf)
        compute_idx(0, idxbufs[0])
        fire(idxbufs[0], rowbufs[0], sems[0])
        for l in range(NUM_LEVELS):
            cur = l & 1
            if l + 1 < NUM_LEVELS:
                compute_idx(l + 1, idxbufs[1 - cur])
                fire(idxbufs[1 - cur], rowbufs[1 - cur], sems[1 - cur])
            drain(idxbufs[cur], rowbufs[cur], sems[cur])
            compute_feat(l, rowbufs[cur])
        pltpu.sync_copy(featbuf, out_hbm.at[pl.ds(base, C), :])
        return c_

    lax.fori_loop(0, NCHUNK, chunk_body, 0)


@functools.partial(jax.jit, static_argnames=())
def _sc_encode(x, tab2):
    mesh = plsc.VectorSubcoreMesh(core_axis_name="c", subcore_axis_name="s")
    kfn = pl.kernel(
        _sc_body,
        out_type=jax.ShapeDtypeStruct((N_POINTS, F_IN), jnp.float32),
        mesh=mesh,
        compiler_params=pltpu.CompilerParams(
            needs_layout_passes=False, use_tc_tiling_on_sc=False),
        scratch_types=[
            pltpu.VMEM((C, 3), jnp.float32),
            pltpu.VMEM((NDMA, 128), jnp.int32),
            pltpu.VMEM((NDMA, 128), jnp.int32),
            pltpu.VMEM((NIDX, 2), jnp.float32),
            pltpu.VMEM((NIDX, 2), jnp.float32),
            pltpu.VMEM((C, F_IN), jnp.float32),
            pltpu.SemaphoreType.DMA,
            pltpu.SemaphoreType.DMA,
        ],
    )
    return kfn(x, tab2)


def _mlp_body(f_ref, w1_ref, b1_ref, w2_ref, b2_ref, out_ref):
    f = f_ref[...]
    h = jnp.dot(f, w1_ref[...], preferred_element_type=jnp.float32)
    h = jnp.maximum(h + b1_ref[...], 0.0)
    o = jnp.dot(h, w2_ref[...], preferred_element_type=jnp.float32)
    o = o + b2_ref[...]
    out_ref[...] = 1.0 / (1.0 + jnp.exp(-o))


def _mlp(feat, W1, b1, W2, b2):
    BN = 4096
    grid = N_POINTS // BN
    return pl.pallas_call(
        _mlp_body,
        grid=(grid,),
        in_specs=[
            pl.BlockSpec((BN, F_IN), lambda i: (i, 0)),
            pl.BlockSpec((F_IN, HIDDEN), lambda i: (0, 0)),
            pl.BlockSpec((1, HIDDEN), lambda i: (0, 0)),
            pl.BlockSpec((HIDDEN, 3), lambda i: (0, 0)),
            pl.BlockSpec((1, 3), lambda i: (0, 0)),
        ],
        out_specs=pl.BlockSpec((BN, 3), lambda i: (i, 0)),
        out_shape=jax.ShapeDtypeStruct((N_POINTS, 3), jnp.float32),
    )(feat, W1, b1, W2, b2)


def kernel(x, tables, W1, b1, W2, b2):
    tab2 = tables.reshape(NUM_LEVELS * T, 2)
    feat = _sc_encode(x, tab2)
    return _mlp(feat, W1, b1.reshape(1, HIDDEN), W2, b2.reshape(1, 3))


# speedup vs baseline: 7.5044x; 7.5044x over previous
import jax
import jax.numpy as jnp
import numpy as np
from jax import lax
from jax.experimental import pallas as pl
from jax.experimental.pallas import tpu as pltpu
from jax.experimental.pallas import tpu_sc as plsc

NUM_LEVELS = 16
BASE_RES = 16
MAX_RES = 2048
LOG2_T = 19
T = 1 << LOG2_T
N_POINTS = 262144
HIDDEN = 32
F_IN = NUM_LEVELS * 2
NROWS = NUM_LEVELS * T
NELEM = NROWS * 2

_B = np.exp((np.log(MAX_RES) - np.log(BASE_RES)) / (NUM_LEVELS - 1))
_RES = [int(np.floor(BASE_RES * (_B ** l))) for l in range(NUM_LEVELS)]

_P1 = np.uint32(2654435761)
_P2 = np.uint32(805459861)
_MASK = np.uint32(T - 1)

NC = 2
NS = 16
NW = NC * NS
PPW = N_POINTS // NW
C = 512
NCHUNK = PPW // C
G = C // 16
NIDX = 8 * C
NDMA = NIDX // 128

CV_TILE = 256
CV_TPW = NELEM // CV_TILE // NW
CV_TB = 32
CV_BLK = CV_TB * CV_TILE
CV_ROWS = CV_BLK // 2
CV_ITER = CV_TPW // CV_TB


def _mesh():
    return plsc.VectorSubcoreMesh(core_axis_name="c", subcore_axis_name="s")


_SC_PARAMS = dict(
    compiler_params=pltpu.CompilerParams(
        needs_layout_passes=False, use_tc_tiling_on_sc=False),
)


def _convert_body(tabflat_hbm, tab2_hbm, inbuf, outbuf):
    wid = lax.axis_index("s") * NC + lax.axis_index("c")
    lane = lax.iota(jnp.int32, 16)
    col0 = lane * 0
    col1 = col0 + 1

    def blk_body(it, c_):
        off = (wid * CV_TPW + it * CV_TB) * CV_TILE
        pltpu.sync_copy(tabflat_hbm.at[pl.ds(off, CV_BLK)], inbuf)

        def t_body(t, c2_):
            tb = t * CV_TILE
            for j in range(8):
                f0 = inbuf[pl.ds(tb + j * 16, 16)]
                f1 = inbuf[pl.ds(tb + 128 + j * 16, 16)]
                row = t * 128 + j * 16 + lane
                plsc.store_scatter(outbuf, [row, col0], f0)
                plsc.store_scatter(outbuf, [row, col1], f1)
            return c2_

        lax.fori_loop(0, CV_TB, t_body, 0)
        pltpu.sync_copy(outbuf, tab2_hbm.at[pl.ds(off // 2, CV_ROWS), :])
        return c_

    lax.fori_loop(0, CV_ITER, blk_body, 0)


def _sc_convert(tabflat):
    kfn = pl.kernel(
        _convert_body,
        out_type=jax.ShapeDtypeStruct((NROWS, 2), jnp.float32),
        mesh=_mesh(),
        scratch_types=[
            pltpu.VMEM((CV_BLK,), jnp.float32),
            pltpu.VMEM((CV_ROWS, 2), jnp.float32),
        ],
        **_SC_PARAMS,
    )
    return kfn(tabflat)


def _sc_body(x_hbm, tab_hbm, out_hbm, xbuf, idx0, idx1, rows0, rows1,
             featbuf, sem0, sem1):
    wid = lax.axis_index("s") * NC + lax.axis_index("c")
    lane = lax.iota(jnp.int32, 16)
    col0 = lane * 0
    col1 = col0 + 1
    col2 = col0 + 2
    idxbufs = (idx0, idx1)
    rowbufs = (rows0, rows1)
    sems = (sem0, sem1)

    def load_xyz(g):
        r16 = g * 16 + lane
        x0 = plsc.load_gather(xbuf, [r16, col0])
        x1 = plsc.load_gather(xbuf, [r16, col1])
        x2 = plsc.load_gather(xbuf, [r16, col2])
        return r16, x0, x1, x2

    def compute_idx(l, ib):
        res = np.float32(_RES[l])
        off = np.int32(l * T)

        def g_body(g, c_):
            _, x0, x1, x2 = load_xyz(g)
            i0 = (x0 * res).astype(jnp.int32)
            i1 = (x1 * res).astype(jnp.int32)
            i2 = (x2 * res).astype(jnp.int32)
            hx0 = i0.astype(jnp.uint32)
            hx1 = hx0 + np.uint32(1)
            u1 = i1.astype(jnp.uint32)
            u2 = i2.astype(jnp.uint32)
            ty0 = u1 * _P1
            ty1 = (u1 + np.uint32(1)) * _P1
            tz0 = u2 * _P2
            tz1 = (u2 + np.uint32(1)) * _P2
            grow = jnp.full((16,), g, jnp.int32)
            lane8 = lane * 8
            for c in range(8):
                hx = hx1 if (c & 1) else hx0
                ty = ty1 if (c & 2) else ty0
                tz = tz1 if (c & 4) else tz0
                h = ((hx ^ ty ^ tz) & _MASK).astype(jnp.int32) + off
                plsc.store_scatter(ib, [grow, lane8 + c], h)
            return c_

        lax.fori_loop(0, G, g_body, 0)

    def fire(ib, rb, sem):
        def j_body(j, c_):
            pltpu.async_copy(tab_hbm.at[ib.at[j]],
                             rb.at[pl.ds(j * 128, 128), :], sem)
            return c_

        lax.fori_loop(0, NDMA, j_body, 0)

    def drain(ib, rb, sem):
        def j_body(j, c_):
            pltpu.make_async_copy(tab_hbm.at[ib.at[j]],
                                  rb.at[pl.ds(j * 128, 128), :], sem).wait()
            return c_

        lax.fori_loop(0, NDMA, j_body, 0)

    def compute_feat(l, rb):
        res = np.float32(_RES[l])
        fcol0 = col0 + (2 * l)
        fcol1 = col0 + (2 * l + 1)

        def g_body(g, c_):
            r16, x0, x1, x2 = load_xyz(g)
            p0 = x0 * res
            p1 = x1 * res
            p2 = x2 * res
            i0 = p0.astype(jnp.int32)
            i1 = p1.astype(jnp.int32)
            i2 = p2.astype(jnp.int32)
            fx = p0 - i0.astype(jnp.float32)
            fy = p1 - i1.astype(jnp.float32)
            fz = p2 - i2.astype(jnp.float32)
            wx = (1.0 - fx, fx)
            wy = (1.0 - fy, fy)
            wz = (1.0 - fz, fz)
            wxy = [wx[cx] * wy[cy] for cy in range(2) for cx in range(2)]
            acc0 = None
            acc1 = None
            rbase = r16 * 8
            for c in range(8):
                w = wxy[c & 3] * wz[(c >> 2) & 1]
                ridx = rbase + c
                f0 = plsc.load_gather(rb, [ridx, col0])
                f1 = plsc.load_gather(rb, [ridx, col1])
                if c == 0:
                    acc0 = w * f0
                    acc1 = w * f1
                else:
                    acc0 = acc0 + w * f0
                    acc1 = acc1 + w * f1
            plsc.store_scatter(featbuf, [r16, fcol0], acc0)
            plsc.store_scatter(featbuf, [r16, fcol1], acc1)
            return c_

        lax.fori_loop(0, G, g_body, 0)

    def chunk_body(ci, c_):
        base = wid * PPW + ci * C
        pltpu.sync_copy(x_hbm.at[pl.ds(base, C), :], xbuf)
        compute_idx(0, idxbufs[0])
        fire(idxbufs[0], rowbufs[0], sems[0])
        for l in range(NUM_LEVELS):
            cur = l & 1
            if l + 1 < NUM_LEVELS:
                compute_idx(l + 1, idxbufs[1 - cur])
                fire(idxbufs[1 - cur], rowbufs[1 - cur], sems[1 - cur])
            drain(idxbufs[cur], rowbufs[cur], sems[cur])
            compute_feat(l, rowbufs[cur])
        pltpu.sync_copy(featbuf, out_hbm.at[pl.ds(base, C), :])
        return c_

    lax.fori_loop(0, NCHUNK, chunk_body, 0)


def _sc_encode(x, tab2):
    kfn = pl.kernel(
        _sc_body,
        out_type=jax.ShapeDtypeStruct((N_POINTS, F_IN), jnp.float32),
        mesh=_mesh(),
        scratch_types=[
            pltpu.VMEM((C, 3), jnp.float32),
            pltpu.VMEM((NDMA, 128), jnp.int32),
            pltpu.VMEM((NDMA, 128), jnp.int32),
            pltpu.VMEM((NIDX, 2), jnp.float32),
            pltpu.VMEM((NIDX, 2), jnp.float32),
            pltpu.VMEM((C, F_IN), jnp.float32),
            pltpu.SemaphoreType.DMA,
            pltpu.SemaphoreType.DMA,
        ],
        **_SC_PARAMS,
    )
    return kfn(x, tab2)


def _mlp_body(f_ref, w1_ref, b1_ref, w2_ref, b2_ref, out_ref):
    f = f_ref[...]
    h = jnp.dot(f, w1_ref[...], preferred_element_type=jnp.float32)
    h = jnp.maximum(h + b1_ref[...], 0.0)
    o = jnp.dot(h, w2_ref[...], preferred_element_type=jnp.float32)
    o = o + b2_ref[...]
    out_ref[...] = 1.0 / (1.0 + jnp.exp(-o))


def _mlp(feat, W1, b1, W2, b2):
    BN = 4096
    grid = N_POINTS // BN
    return pl.pallas_call(
        _mlp_body,
        grid=(grid,),
        in_specs=[
            pl.BlockSpec((BN, F_IN), lambda i: (i, 0)),
            pl.BlockSpec((F_IN, HIDDEN), lambda i: (0, 0)),
            pl.BlockSpec((1, HIDDEN), lambda i: (0, 0)),
            pl.BlockSpec((HIDDEN, 3), lambda i: (0, 0)),
            pl.BlockSpec((1, 3), lambda i: (0, 0)),
        ],
        out_specs=pl.BlockSpec((BN, 3), lambda i: (i, 0)),
        out_shape=jax.ShapeDtypeStruct((N_POINTS, 3), jnp.float32),
    )(feat, W1, b1, W2, b2)


def kernel(x, tables, W1, b1, W2, b2):
    tabflat = tables.reshape(NUM_LEVELS, T // 128, 128, 2)
    tabflat = tabflat.transpose(0, 1, 3, 2).reshape(NELEM)
    tab2 = _sc_convert(tabflat)
    feat = _sc_encode(x, tab2)
    return _mlp(feat, W1, b1.reshape(1, HIDDEN), W2, b2.reshape(1, 3))
